# baseline (speedup 1.0000x reference)
import functools

import numpy as np

import jax
import jax.numpy as jnp
from jax import lax
from jax.experimental import pallas as pl
from jax.experimental.pallas import tpu as pltpu

S = 30.0
M = 0.7
H_PARAM = 0.5
ADA_EPS = 1e-6
T_ALPHA = 0.01
GEM_P = 3.0
GEM_EPS = 1e-6
BN_EPS = 1e-5

_VMEM_LIMIT = 56 * 1024 * 1024


def _make_stem_kernel(hw):
    inv_hw = 1.0 / float(hw)

    def _body(x_ref, w_ref, o_ref):
        x = x_ref[0]
        ones = jnp.ones((1, x.shape[1]), jnp.bfloat16)
        x4 = jnp.concatenate([x, ones], axis=0)
        feat = jnp.dot(w_ref[...], x4, preferred_element_type=jnp.float32)
        m = jnp.maximum(feat, GEM_EPS)
        acc = jnp.sum(m * m * m, axis=1, keepdims=True)
        mean = acc * inv_hw
        o_ref[0] = jnp.exp(jnp.log(mean) * (1.0 / GEM_P))

    return _body


def _stem_gem(x_bc_hw, w4):
    B, C, HW = x_bc_hw.shape
    F = w4.shape[0]

    out = pl.pallas_call(
        _make_stem_kernel(HW),
        grid=(B,),
        in_specs=[
            pl.BlockSpec((1, C, HW), lambda b: (b, 0, 0)),
            pl.BlockSpec((F, C + 1), lambda b: (0, 0)),
        ],
        out_specs=pl.BlockSpec((1, F, 1), lambda b: (b, 0, 0)),
        out_shape=jax.ShapeDtypeStruct((B, F, 1), jnp.float32),
        compiler_params=pltpu.CompilerParams(
            dimension_semantics=("parallel",),
            vmem_limit_bytes=_VMEM_LIMIT,
        ),
        cost_estimate=pl.CostEstimate(
            flops=int(2 * B * HW * (C + 1) * F + 4 * B * HW * F),
            transcendentals=int(2 * B * F),
            bytes_accessed=int(4 * (B * C * HW + F * (C + 1) + B * F)),
        ),
    )(x_bc_hw, w4)
    return out.reshape(B, F)


def _make_decoder_kernel(tnc):
    def _body(pooled_ref, wneck_ref, bneck_ref, gamma_ref, beta_ref,
              rmean_ref, rvar_ref, buf_ref, label_ref, wdec_ref,
              o_ref, stats_ref, z_sc, margin_sc):
        j = pl.program_id(0)
        nb = pooled_ref.shape[0]

        @pl.when(j == 0)
        def _prologue():
            y = (jnp.dot(pooled_ref[...], wneck_ref[...],
                         preferred_element_type=jnp.float32) + bneck_ref[...])
            y = ((y - rmean_ref[...]) * lax.rsqrt(rvar_ref[...] + BN_EPS)
                 * gamma_ref[...] + beta_ref[...])
            y = 0.5 * y * (1.0 + lax.erf(y * 0.7071067811865476))

            norm = jnp.maximum(jnp.sqrt(jnp.sum(y * y, axis=1, keepdims=True)),
                               ADA_EPS)
            z_sc[...] = y / norm

            bmean = jnp.mean(norm, axis=0, keepdims=True)
            diff = norm - bmean
            denom = float(max(nb - 1, 1))
            bstd = jnp.sqrt(jnp.sum(diff * diff, axis=0, keepdims=True) / denom)
            new_mean = (1.0 - T_ALPHA) * buf_ref[:, 0:1] + T_ALPHA * bmean
            new_std = (1.0 - T_ALPHA) * buf_ref[:, 1:2] + T_ALPHA * bstd
            stats_ref[:, 0:1] = new_mean
            stats_ref[:, 1:2] = new_std
            margin_sc[...] = M + H_PARAM * (norm - new_mean) / (new_std + ADA_EPS)

        w = wdec_ref[...].astype(jnp.float32)
        inv_wn = lax.rsqrt(jnp.maximum(jnp.sum(w * w, axis=1, keepdims=True),
                                       1e-24))
        cosine = lax.dot_general(z_sc[...], w * inv_wn,
                                 (((1,), (1,)), ((), ())),
                                 preferred_element_type=jnp.float32)
        cosine = jnp.clip(cosine, -1.0 + ADA_EPS, 1.0 - ADA_EPS)

        cls = lax.broadcasted_iota(jnp.int32, (nb, tnc), 1) + j * tnc
        m_ps = margin_sc[...]
        sin_t = jnp.sqrt(jnp.maximum(1.0 - cosine * cosine, 0.0))
        target = cosine * jnp.cos(m_ps) - sin_t * jnp.sin(m_ps)
        o_ref[...] = jnp.where(cls == label_ref[...], target, cosine) * S

    return _body


def _neck_decoder(pooled, w_neck, b_neck, bn_gamma, bn_beta, bn_mean, bn_var,
                  ada_buffers, labels_loc, w_dec_loc):
    B, F = pooled.shape
    E = w_neck.shape[1]
    nc_loc = w_dec_loc.shape[0]
    tnc = nc_loc
    for t in (1024, 512, 256, 128):
        if nc_loc % t == 0:
            tnc = t
            break
    nj = nc_loc // tnc

    args = (pooled, w_neck, b_neck, bn_gamma, bn_beta, bn_mean, bn_var,
            ada_buffers, labels_loc, w_dec_loc)
    in_specs = [
        pl.BlockSpec((B, F), lambda j: (0, 0)),
        pl.BlockSpec((F, E), lambda j: (0, 0)),
        pl.BlockSpec((1, E), lambda j: (0, 0)),
        pl.BlockSpec((1, E), lambda j: (0, 0)),
        pl.BlockSpec((1, E), lambda j: (0, 0)),
        pl.BlockSpec((1, E), lambda j: (0, 0)),
        pl.BlockSpec((1, E), lambda j: (0, 0)),
        pl.BlockSpec((1, 2), lambda j: (0, 0)),
        pl.BlockSpec((B, 1), lambda j: (0, 0)),
        pl.BlockSpec((tnc, E), lambda j: (j, 0)),
    ]
    out_specs = (
        pl.BlockSpec((B, tnc), lambda j: (0, j)),
        pl.BlockSpec((1, 2), lambda j: (0, 0)),
    )
    out_shape = (
        jax.ShapeDtypeStruct((B, nc_loc), jnp.float32),
        jax.ShapeDtypeStruct((1, 2), jnp.float32),
    )
    return pl.pallas_call(
        _make_decoder_kernel(tnc),
        grid=(nj,),
        in_specs=in_specs,
        out_specs=out_specs,
        out_shape=out_shape,
        scratch_shapes=[pltpu.VMEM((B, E), jnp.float32),
                        pltpu.VMEM((B, 1), jnp.float32)],
        compiler_params=pltpu.CompilerParams(
            dimension_semantics=("arbitrary",),
            vmem_limit_bytes=_VMEM_LIMIT,
        ),
        cost_estimate=pl.CostEstimate(
            flops=int(2 * B * F * E + 3 * nc_loc * E + 2 * B * E * nc_loc
                      + 10 * B * nc_loc),
            transcendentals=int(2 * B * E + nc_loc + 8 * B),
            bytes_accessed=int(4 * (B * F + F * E + 6 * E + B + nc_loc * E
                                    + B * nc_loc + 4)),
        ),
    )(*args)


def _fwd(x_sh, labels, w4, w_neck, b_neck, bn_gamma, bn_beta, bn_mean, bn_var,
         w_dec_sh, ada_buffers, *, nc_loc):
    pooled = _stem_gem(x_sh, w4)
    pooled = lax.all_gather(pooled, 'd', axis=0, tiled=True)
    didx = lax.axis_index('d')
    labels_loc = (labels - didx * nc_loc).reshape(-1, 1)
    return _neck_decoder(pooled, w_neck, b_neck, bn_gamma, bn_beta, bn_mean,
                         bn_var, ada_buffers, labels_loc, w_dec_sh)


def kernel(x, labels, w_stem, b_stem, w_neck, b_neck, bn_gamma, bn_beta,
           bn_mean, bn_var, w_dec, ada_buffers):
    B, C, Himg, Wimg = x.shape
    F = w_stem.shape[0]
    E = w_neck.shape[1]
    NC = w_dec.shape[0]
    x3 = x.reshape(B, C, Himg * Wimg)
    if labels is None:
        labels = jnp.full((B,), -1, dtype=jnp.int32)
    labels = labels.astype(jnp.int32)
    x3 = x3.astype(jnp.bfloat16)
    w4 = jnp.concatenate([w_stem, b_stem.reshape(F, 1)],
                         axis=1).astype(jnp.bfloat16)

    devs = jax.devices()
    ndev = max(n for n in (8, 4, 2, 1) if n <= len(devs) and B % n == 0)
    nc_pad = -(-NC // (128 * ndev)) * (128 * ndev)
    w_dec_p = (jnp.pad(w_dec, ((0, nc_pad - NC), (0, 0)))
               if nc_pad != NC else w_dec).astype(jnp.bfloat16)

    mesh = jax.sharding.Mesh(np.array(devs[:ndev]), ('d',))
    P = jax.sharding.PartitionSpec
    fwd = jax.shard_map(
        functools.partial(_fwd, nc_loc=nc_pad // ndev),
        mesh=mesh,
        in_specs=(P('d'), P(), P(), P(), P(), P(), P(), P(), P(), P('d'), P()),
        out_specs=(P(None, 'd'), P()),
        check_vma=False,
    )
    logits, new_buffers = fwd(
        x3, labels, w4, w_neck, b_neck.reshape(1, E), bn_gamma.reshape(1, E),
        bn_beta.reshape(1, E), bn_mean.reshape(1, E), bn_var.reshape(1, E),
        w_dec_p, ada_buffers.reshape(1, 2))
    return logits[:, :NC], new_buffers


# speedup vs baseline: 1.0696x; 1.0696x over previous
import jax
import jax.numpy as jnp
from jax import lax
from jax.experimental import pallas as pl
from jax.experimental.pallas import tpu as pltpu

S = 30.0
M = 0.7
H_PARAM = 0.5
ADA_EPS = 1e-6
T_ALPHA = 0.01
GEM_P = 3.0
GEM_EPS = 1e-6
BN_EPS = 1e-5

_VMEM_LIMIT = 56 * 1024 * 1024
_TAIL = 256


def _make_stem_kernel(hw):
    inv_hw = 1.0 / float(hw)

    def _body(x_ref, w_ref, o_ref):
        x = x_ref[0]
        ones = jnp.ones((1, x.shape[1]), jnp.bfloat16)
        x4 = jnp.concatenate([x, ones], axis=0)
        feat = jnp.dot(w_ref[...], x4, preferred_element_type=jnp.float32)
        m = jnp.maximum(feat, GEM_EPS)
        acc = jnp.sum(m * m * m, axis=1, keepdims=True)
        mean = acc * inv_hw
        o_ref[0] = jnp.exp(jnp.log(mean) * (1.0 / GEM_P))

    return _body


def _stem_gem(x_bc_hw, w4):
    B, C, HW = x_bc_hw.shape
    F = w4.shape[0]

    out = pl.pallas_call(
        _make_stem_kernel(HW),
        grid=(B,),
        in_specs=[
            pl.BlockSpec((1, C, HW), lambda b: (b, 0, 0)),
            pl.BlockSpec((F, C + 1), lambda b: (0, 0)),
        ],
        out_specs=pl.BlockSpec((1, F, 1), lambda b: (b, 0, 0)),
        out_shape=jax.ShapeDtypeStruct((B, F, 1), jnp.float32),
        compiler_params=pltpu.CompilerParams(
            dimension_semantics=("parallel",),
            vmem_limit_bytes=_VMEM_LIMIT,
        ),
        cost_estimate=pl.CostEstimate(
            flops=int(2 * B * HW * (C + 1) * F + 4 * B * HW * F),
            transcendentals=int(2 * B * F),
            bytes_accessed=int(2 * B * C * HW + 2 * F * (C + 1) + 4 * B * F),
        ),
    )(x_bc_hw, w4)
    return out.reshape(B, F)


def _make_decoder_kernel(tnc):
    def _body(pooled_ref, wneck_ref, bneck_ref, gamma_ref, beta_ref,
              rmean_ref, rvar_ref, buf_ref, label_ref, wdec_ref,
              o_ref, stats_ref, z_sc, margin_sc):
        j = pl.program_id(0)
        nb = pooled_ref.shape[0]

        @pl.when(j == 0)
        def _prologue():
            y = (jnp.dot(pooled_ref[...], wneck_ref[...],
                         preferred_element_type=jnp.float32) + bneck_ref[...])
            y = ((y - rmean_ref[...]) * lax.rsqrt(rvar_ref[...] + BN_EPS)
                 * gamma_ref[...] + beta_ref[...])
            y = 0.5 * y * (1.0 + lax.erf(y * 0.7071067811865476))

            norm = jnp.maximum(jnp.sqrt(jnp.sum(y * y, axis=1, keepdims=True)),
                               ADA_EPS)
            z_sc[...] = y / norm

            bmean = jnp.mean(norm, axis=0, keepdims=True)
            diff = norm - bmean
            denom = float(max(nb - 1, 1))
            bstd = jnp.sqrt(jnp.sum(diff * diff, axis=0, keepdims=True) / denom)
            new_mean = (1.0 - T_ALPHA) * buf_ref[:, 0:1] + T_ALPHA * bmean
            new_std = (1.0 - T_ALPHA) * buf_ref[:, 1:2] + T_ALPHA * bstd
            stats_ref[:, 0:1] = new_mean
            stats_ref[:, 1:2] = new_std
            margin_sc[...] = M + H_PARAM * (norm - new_mean) / (new_std + ADA_EPS)

        w = wdec_ref[...].astype(jnp.float32)
        inv_wn = lax.rsqrt(jnp.maximum(jnp.sum(w * w, axis=1, keepdims=True),
                                       1e-24))
        cosine = lax.dot_general(z_sc[...], w * inv_wn,
                                 (((1,), (1,)), ((), ())),
                                 preferred_element_type=jnp.float32)
        cosine = jnp.clip(cosine, -1.0 + ADA_EPS, 1.0 - ADA_EPS)

        cls = lax.broadcasted_iota(jnp.int32, (nb, tnc), 1) + j * tnc
        m_ps = margin_sc[...]
        sin_t = jnp.sqrt(jnp.maximum(1.0 - cosine * cosine, 0.0))
        target = cosine * jnp.cos(m_ps) - sin_t * jnp.sin(m_ps)
        o_ref[...] = jnp.where(cls == label_ref[...], target, cosine) * S

    return _body


def _neck_decoder(pooled, w_neck, b_neck, bn_gamma, bn_beta, bn_mean, bn_var,
                  ada_buffers, labels, w_dec):
    B, F = pooled.shape
    E = w_neck.shape[1]
    nc = w_dec.shape[0]
    tnc = nc
    for t in (1024, 512, 256, 128):
        if nc % t == 0:
            tnc = t
            break
    nj = nc // tnc

    args = (pooled, w_neck, b_neck, bn_gamma, bn_beta, bn_mean, bn_var,
            ada_buffers, labels, w_dec)
    in_specs = [
        pl.BlockSpec((B, F), lambda j: (0, 0)),
        pl.BlockSpec((F, E), lambda j: (0, 0)),
        pl.BlockSpec((1, E), lambda j: (0, 0)),
        pl.BlockSpec((1, E), lambda j: (0, 0)),
        pl.BlockSpec((1, E), lambda j: (0, 0)),
        pl.BlockSpec((1, E), lambda j: (0, 0)),
        pl.BlockSpec((1, E), lambda j: (0, 0)),
        pl.BlockSpec((1, 2), lambda j: (0, 0)),
        pl.BlockSpec((B, 1), lambda j: (0, 0)),
        pl.BlockSpec((tnc, E), lambda j: (j, 0)),
    ]
    out_specs = (
        pl.BlockSpec((B, tnc), lambda j: (0, j)),
        pl.BlockSpec((1, 2), lambda j: (0, 0)),
    )
    out_shape = (
        jax.ShapeDtypeStruct((B, nc), jnp.float32),
        jax.ShapeDtypeStruct((1, 2), jnp.float32),
    )
    return pl.pallas_call(
        _make_decoder_kernel(tnc),
        grid=(nj,),
        in_specs=in_specs,
        out_specs=out_specs,
        out_shape=out_shape,
        scratch_shapes=[pltpu.VMEM((B, E), jnp.float32),
                        pltpu.VMEM((B, 1), jnp.float32)],
        compiler_params=pltpu.CompilerParams(
            dimension_semantics=("arbitrary",),
            vmem_limit_bytes=_VMEM_LIMIT,
        ),
        cost_estimate=pl.CostEstimate(
            flops=int(2 * B * F * E + 3 * nc * E + 2 * B * E * nc
                      + 10 * B * nc),
            transcendentals=int(2 * B * E + nc + 8 * B),
            bytes_accessed=int(4 * (B * F + F * E + 6 * E + B + B * nc + 4)
                               + 2 * nc * E),
        ),
    )(*args)


def kernel(x, labels, w_stem, b_stem, w_neck, b_neck, bn_gamma, bn_beta,
           bn_mean, bn_var, w_dec, ada_buffers):
    B, C, Himg, Wimg = x.shape
    F = w_stem.shape[0]
    E = w_neck.shape[1]
    NC = w_dec.shape[0]
    if labels is None:
        labels = jnp.full((B,), -1, dtype=jnp.int32)
    labels = labels.astype(jnp.int32).reshape(B, 1)
    x3 = x.reshape(B, C, Himg * Wimg).astype(jnp.bfloat16)
    w4 = jnp.concatenate([w_stem, b_stem.reshape(F, 1)],
                         axis=1).astype(jnp.bfloat16)
    nc_pad = -(-NC // 128) * 128
    w_dec_p = (jnp.pad(w_dec, ((0, nc_pad - NC), (0, 0)))
               if nc_pad != NC else w_dec).astype(jnp.bfloat16)

    pooled = _stem_gem(x3, w4)
    logits, new_buffers = _neck_decoder(
        pooled, w_neck, b_neck.reshape(1, E), bn_gamma.reshape(1, E),
        bn_beta.reshape(1, E), bn_mean.reshape(1, E), bn_var.reshape(1, E),
        ada_buffers.reshape(1, 2), labels, w_dec_p)
    return logits[:, :NC], new_buffers


# speedup vs baseline: 1.0981x; 1.0267x over previous
import jax
import jax.numpy as jnp
from jax import lax
from jax.experimental import pallas as pl
from jax.experimental.pallas import tpu as pltpu

S = 30.0
M = 0.7
H_PARAM = 0.5
ADA_EPS = 1e-6
T_ALPHA = 0.01
GEM_P = 3.0
GEM_EPS = 1e-6
BN_EPS = 1e-5

_VMEM_LIMIT = 56 * 1024 * 1024
_TAIL = 256


def _make_stem_kernel(hw):
    inv_hw = 1.0 / float(hw)

    def _body(x_ref, w_ref, o_ref):
        x = x_ref[0]
        ones = jnp.ones((1, x.shape[1]), jnp.float32)
        x4 = jnp.concatenate([x, ones], axis=0)
        feat = jnp.dot(w_ref[...], x4, preferred_element_type=jnp.float32)
        m = jnp.maximum(feat, GEM_EPS)
        acc = jnp.sum(m * m * m, axis=1, keepdims=True)
        mean = acc * inv_hw
        o_ref[0] = jnp.exp(jnp.log(mean) * (1.0 / GEM_P))

    return _body


def _stem_gem(x_bc_hw, w4):
    B, C, HW = x_bc_hw.shape
    F = w4.shape[0]

    out = pl.pallas_call(
        _make_stem_kernel(HW),
        grid=(B,),
        in_specs=[
            pl.BlockSpec((1, C, HW), lambda b: (b, 0, 0)),
            pl.BlockSpec((F, C + 1), lambda b: (0, 0)),
        ],
        out_specs=pl.BlockSpec((1, F, 1), lambda b: (b, 0, 0)),
        out_shape=jax.ShapeDtypeStruct((B, F, 1), jnp.float32),
        compiler_params=pltpu.CompilerParams(
            dimension_semantics=("parallel",),
            vmem_limit_bytes=_VMEM_LIMIT,
        ),
        cost_estimate=pl.CostEstimate(
            flops=int(2 * B * HW * (C + 1) * F + 4 * B * HW * F),
            transcendentals=int(2 * B * F),
            bytes_accessed=int(2 * B * C * HW + 2 * F * (C + 1) + 4 * B * F),
        ),
    )(x_bc_hw, w4)
    return out.reshape(B, F)


def _make_decoder_kernel(tnc):
    def _body(pooled_ref, wneck_ref, bneck_ref, gamma_ref, beta_ref,
              rmean_ref, rvar_ref, buf_ref, label_ref, wdec_ref,
              o_ref, stats_ref, z_sc, margin_sc):
        j = pl.program_id(0)
        nb = pooled_ref.shape[0]

        @pl.when(j == 0)
        def _prologue():
            y = (jnp.dot(pooled_ref[...], wneck_ref[...],
                         preferred_element_type=jnp.float32) + bneck_ref[...])
            y = ((y - rmean_ref[...]) * lax.rsqrt(rvar_ref[...] + BN_EPS)
                 * gamma_ref[...] + beta_ref[...])
            y = 0.5 * y * (1.0 + lax.erf(y * 0.7071067811865476))

            norm = jnp.maximum(jnp.sqrt(jnp.sum(y * y, axis=1, keepdims=True)),
                               ADA_EPS)
            z_sc[...] = y / norm

            bmean = jnp.mean(norm, axis=0, keepdims=True)
            diff = norm - bmean
            denom = float(max(nb - 1, 1))
            bstd = jnp.sqrt(jnp.sum(diff * diff, axis=0, keepdims=True) / denom)
            new_mean = (1.0 - T_ALPHA) * buf_ref[:, 0:1] + T_ALPHA * bmean
            new_std = (1.0 - T_ALPHA) * buf_ref[:, 1:2] + T_ALPHA * bstd
            stats_ref[:, 0:1] = new_mean
            stats_ref[:, 1:2] = new_std
            margin_sc[...] = M + H_PARAM * (norm - new_mean) / (new_std + ADA_EPS)

        w = wdec_ref[...]
        inv_wn = lax.rsqrt(jnp.maximum(jnp.sum(w * w, axis=1, keepdims=True),
                                       1e-24))
        cosine = lax.dot_general(z_sc[...], w * inv_wn,
                                 (((1,), (1,)), ((), ())),
                                 preferred_element_type=jnp.float32)
        cosine = jnp.clip(cosine, -1.0 + ADA_EPS, 1.0 - ADA_EPS)

        cls = lax.broadcasted_iota(jnp.int32, (nb, tnc), 1) + j * tnc
        m_ps = margin_sc[...]
        sin_t = jnp.sqrt(jnp.maximum(1.0 - cosine * cosine, 0.0))
        target = cosine * jnp.cos(m_ps) - sin_t * jnp.sin(m_ps)
        o_ref[...] = jnp.where(cls == label_ref[...], target, cosine) * S

    return _body


def _neck_decoder(pooled, w_neck, b_neck, bn_gamma, bn_beta, bn_mean, bn_var,
                  ada_buffers, labels, w_dec):
    B, F = pooled.shape
    E = w_neck.shape[1]
    nc = w_dec.shape[0]
    tnc = nc
    for t in (1024, 512, 256, 128):
        if nc % t == 0:
            tnc = t
            break
    nj = nc // tnc

    args = (pooled, w_neck, b_neck, bn_gamma, bn_beta, bn_mean, bn_var,
            ada_buffers, labels, w_dec)
    in_specs = [
        pl.BlockSpec((B, F), lambda j: (0, 0)),
        pl.BlockSpec((F, E), lambda j: (0, 0)),
        pl.BlockSpec((1, E), lambda j: (0, 0)),
        pl.BlockSpec((1, E), lambda j: (0, 0)),
        pl.BlockSpec((1, E), lambda j: (0, 0)),
        pl.BlockSpec((1, E), lambda j: (0, 0)),
        pl.BlockSpec((1, E), lambda j: (0, 0)),
        pl.BlockSpec((1, 2), lambda j: (0, 0)),
        pl.BlockSpec((B, 1), lambda j: (0, 0)),
        pl.BlockSpec((tnc, E), lambda j: (j, 0)),
    ]
    out_specs = (
        pl.BlockSpec((B, tnc), lambda j: (0, j)),
        pl.BlockSpec((1, 2), lambda j: (0, 0)),
    )
    out_shape = (
        jax.ShapeDtypeStruct((B, nc), jnp.float32),
        jax.ShapeDtypeStruct((1, 2), jnp.float32),
    )
    return pl.pallas_call(
        _make_decoder_kernel(tnc),
        grid=(nj,),
        in_specs=in_specs,
        out_specs=out_specs,
        out_shape=out_shape,
        scratch_shapes=[pltpu.VMEM((B, E), jnp.float32),
                        pltpu.VMEM((B, 1), jnp.float32)],
        compiler_params=pltpu.CompilerParams(
            dimension_semantics=("arbitrary",),
            vmem_limit_bytes=_VMEM_LIMIT,
        ),
        cost_estimate=pl.CostEstimate(
            flops=int(2 * B * F * E + 3 * nc * E + 2 * B * E * nc
                      + 10 * B * nc),
            transcendentals=int(2 * B * E + nc + 8 * B),
            bytes_accessed=int(4 * (B * F + F * E + 6 * E + B + B * nc + 4)
                               + 2 * nc * E),
        ),
    )(*args)


def kernel(x, labels, w_stem, b_stem, w_neck, b_neck, bn_gamma, bn_beta,
           bn_mean, bn_var, w_dec, ada_buffers):
    B, C, Himg, Wimg = x.shape
    F = w_stem.shape[0]
    E = w_neck.shape[1]
    NC = w_dec.shape[0]
    if labels is None:
        labels = jnp.full((B,), -1, dtype=jnp.int32)
    labels = labels.astype(jnp.int32).reshape(B, 1)
    x3 = x.reshape(B, C, Himg * Wimg)
    w4 = jnp.concatenate([w_stem, b_stem.reshape(F, 1)], axis=1)
    nc_pad = -(-NC // 128) * 128
    w_dec_p = (jnp.pad(w_dec, ((0, nc_pad - NC), (0, 0)))
               if nc_pad != NC else w_dec)

    pooled = _stem_gem(x3, w4)
    logits, new_buffers = _neck_decoder(
        pooled, w_neck, b_neck.reshape(1, E), bn_gamma.reshape(1, E),
        bn_beta.reshape(1, E), bn_mean.reshape(1, E), bn_var.reshape(1, E),
        ada_buffers.reshape(1, 2), labels, w_dec_p)
    return logits[:, :NC], new_buffers


# speedup vs baseline: 1.0982x; 1.0001x over previous
import jax
import jax.numpy as jnp
from jax import lax
from jax.experimental import pallas as pl
from jax.experimental.pallas import tpu as pltpu

S = 30.0
M = 0.7
H_PARAM = 0.5
ADA_EPS = 1e-6
T_ALPHA = 0.01
GEM_P = 3.0
GEM_EPS = 1e-6
BN_EPS = 1e-5

_VMEM_LIMIT = 56 * 1024 * 1024


def _make_stem_kernel(hw):
    inv_hw = 1.0 / float(hw)

    def _body(x_ref, w_ref, o_ref):
        x = x_ref[0]
        ones = jnp.ones((1, x.shape[1]), jnp.float32)
        x4 = jnp.concatenate([x, ones], axis=0)
        feat = jnp.dot(w_ref[...], x4, preferred_element_type=jnp.float32)
        m = jnp.maximum(feat, GEM_EPS)
        acc = jnp.sum(m * m * m, axis=1, keepdims=True)
        mean = acc * inv_hw
        o_ref[0] = jnp.exp(jnp.log(mean) * (1.0 / GEM_P))

    return _body


def _stem_gem(x_bc_hw, w40):
    B, C, HW = x_bc_hw.shape
    F, K = w40.shape

    out = pl.pallas_call(
        _make_stem_kernel(HW),
        grid=(B,),
        in_specs=[
            pl.BlockSpec((1, C, HW), lambda b: (b, 0, 0)),
            pl.BlockSpec((F, K), lambda b: (0, 0)),
        ],
        out_specs=pl.BlockSpec((1, F, 1), lambda b: (b, 0, 0)),
        out_shape=jax.ShapeDtypeStruct((B, F, 1), jnp.float32),
        compiler_params=pltpu.CompilerParams(
            dimension_semantics=("parallel",),
            vmem_limit_bytes=_VMEM_LIMIT,
        ),
        cost_estimate=pl.CostEstimate(
            flops=int(2 * B * HW * (C + 1) * F + 4 * B * HW * F),
            transcendentals=int(2 * B * F),
            bytes_accessed=int(4 * (B * C * HW + F * (C + 1) + B * F)),
        ),
    )(x_bc_hw, w40)
    return out.reshape(B, F)


def _make_decoder_kernel(tnc):
    def _body(pooled_ref, wneck_ref, bneck_ref, gamma_ref, beta_ref,
              rmean_ref, rvar_ref, buf_ref, label_ref, wdec_ref,
              o_ref, stats_ref, z_sc, margin_sc):
        j = pl.program_id(0)
        nb = pooled_ref.shape[0]

        @pl.when(j == 0)
        def _prologue():
            y = (jnp.dot(pooled_ref[...], wneck_ref[...],
                         preferred_element_type=jnp.float32) + bneck_ref[...])
            y = ((y - rmean_ref[...]) * lax.rsqrt(rvar_ref[...] + BN_EPS)
                 * gamma_ref[...] + beta_ref[...])
            y = 0.5 * y * (1.0 + lax.erf(y * 0.7071067811865476))

            norm = jnp.maximum(jnp.sqrt(jnp.sum(y * y, axis=1, keepdims=True)),
                               ADA_EPS)
            z_sc[...] = y / norm

            bmean = jnp.mean(norm, axis=0, keepdims=True)
            diff = norm - bmean
            denom = float(max(nb - 1, 1))
            bstd = jnp.sqrt(jnp.sum(diff * diff, axis=0, keepdims=True) / denom)
            new_mean = (1.0 - T_ALPHA) * buf_ref[:, 0:1] + T_ALPHA * bmean
            new_std = (1.0 - T_ALPHA) * buf_ref[:, 1:2] + T_ALPHA * bstd
            stats_ref[:, 0:1] = new_mean
            stats_ref[:, 1:2] = new_std
            margin_sc[...] = M + H_PARAM * (norm - new_mean) / (new_std + ADA_EPS)

        w = wdec_ref[...]
        inv_wn = lax.rsqrt(jnp.maximum(jnp.sum(w * w, axis=1, keepdims=True),
                                       1e-24))
        cosine = lax.dot_general(z_sc[...], w * inv_wn,
                                 (((1,), (1,)), ((), ())),
                                 preferred_element_type=jnp.float32)
        cosine = jnp.clip(cosine, -1.0 + ADA_EPS, 1.0 - ADA_EPS)

        cls = lax.broadcasted_iota(jnp.int32, (nb, tnc), 1) + j * tnc
        m_ps = margin_sc[...]
        sin_t = jnp.sqrt(jnp.maximum(1.0 - cosine * cosine, 0.0))
        target = cosine * jnp.cos(m_ps) - sin_t * jnp.sin(m_ps)
        o_ref[...] = jnp.where(cls == label_ref[...], target, cosine) * S

    return _body


def _neck_decoder(pooled, w_neck, b_neck, bn_gamma, bn_beta, bn_mean, bn_var,
                  ada_buffers, labels, w_dec):
    B, F = pooled.shape
    E = w_neck.shape[1]
    nc = w_dec.shape[0]
    tnc = nc
    for t in (1024, 512, 256, 128):
        if nc % t == 0:
            tnc = t
            break
    nj = nc // tnc

    args = (pooled, w_neck, b_neck, bn_gamma, bn_beta, bn_mean, bn_var,
            ada_buffers, labels, w_dec)
    in_specs = [
        pl.BlockSpec((B, F), lambda j: (0, 0)),
        pl.BlockSpec((F, E), lambda j: (0, 0)),
        pl.BlockSpec((1, E), lambda j: (0, 0)),
        pl.BlockSpec((1, E), lambda j: (0, 0)),
        pl.BlockSpec((1, E), lambda j: (0, 0)),
        pl.BlockSpec((1, E), lambda j: (0, 0)),
        pl.BlockSpec((1, E), lambda j: (0, 0)),
        pl.BlockSpec((1, 2), lambda j: (0, 0)),
        pl.BlockSpec((B, 1), lambda j: (0, 0)),
        pl.BlockSpec((tnc, E), lambda j: (j, 0)),
    ]
    out_specs = (
        pl.BlockSpec((B, tnc), lambda j: (0, j)),
        pl.BlockSpec((1, 2), lambda j: (0, 0)),
    )
    out_shape = (
        jax.ShapeDtypeStruct((B, nc), jnp.float32),
        jax.ShapeDtypeStruct((1, 2), jnp.float32),
    )
    return pl.pallas_call(
        _make_decoder_kernel(tnc),
        grid=(nj,),
        in_specs=in_specs,
        out_specs=out_specs,
        out_shape=out_shape,
        scratch_shapes=[pltpu.VMEM((B, E), jnp.float32),
                        pltpu.VMEM((B, 1), jnp.float32)],
        compiler_params=pltpu.CompilerParams(
            dimension_semantics=("arbitrary",),
            vmem_limit_bytes=_VMEM_LIMIT,
        ),
        cost_estimate=pl.CostEstimate(
            flops=int(2 * B * F * E + 3 * nc * E + 2 * B * E * nc
                      + 10 * B * nc),
            transcendentals=int(2 * B * E + nc + 8 * B),
            bytes_accessed=int(4 * (B * F + F * E + 6 * E + B + B * nc + 4
                                    + nc * E)),
        ),
    )(*args)


def kernel(x, labels, w_stem, b_stem, w_neck, b_neck, bn_gamma, bn_beta,
           bn_mean, bn_var, w_dec, ada_buffers):
    B, C, Himg, Wimg = x.shape
    F = w_stem.shape[0]
    E = w_neck.shape[1]
    NC = w_dec.shape[0]
    if labels is None:
        labels = jnp.full((B,), -1, dtype=jnp.int32)
    labels = labels.astype(jnp.int32).reshape(B, 1)
    x3 = x.reshape(B, C, Himg * Wimg)
    w40 = jnp.concatenate([w_stem, b_stem.reshape(F, 1)], axis=1)
    nc_pad = -(-NC // 128) * 128
    w_dec_p = (jnp.pad(w_dec, ((0, nc_pad - NC), (0, 0)))
               if nc_pad != NC else w_dec)

    pooled = _stem_gem(x3, w40)
    logits, new_buffers = _neck_decoder(
        pooled, w_neck, b_neck.reshape(1, E), bn_gamma.reshape(1, E),
        bn_beta.reshape(1, E), bn_mean.reshape(1, E), bn_var.reshape(1, E),
        ada_buffers.reshape(1, 2), labels, w_dec_p)
    return logits[:, :NC], new_buffers
